# Initial kernel scaffold; baseline (speedup 1.0000x reference)
import jax
import jax.numpy as jnp
from jax.experimental import pallas as pl


def kernel(user_id_idx, user_age_idx, user_gender_idx, user_city_idx, item_id_idx, item_cate_idx, item_brand_idx, item_price_idx, user_emb_user_id, user_emb_user_age, user_emb_user_gender, user_emb_user_city, item_emb_item_id, item_emb_item_cate, item_emb_item_brand, item_emb_item_price, uW1, ub1, uW2, ub2, iW1, ib1, iW2, ib2, oW, ob):
    raise NotImplementedError("write your pallas kernel here")


# speedup vs baseline: 1.4508x; 1.4508x over previous
import functools

import jax
import jax.numpy as jnp
from jax import lax
from jax.experimental import pallas as pl
from jax.experimental.pallas import tpu as pltpu
from jax.experimental.pallas import tpu_sc as plsc

B = 16384
EMB = 32
NC = 2
NS = 16
NW = NC * NS
BPW = B // NW
CHUNK = 128
NCHUNK = BPW // CHUNK
IDX_ROWS = B // CHUNK


def _sc_gather_body(*refs):
    tabs = refs[0:8]
    idxs = refs[8:16]
    outs = refs[16:24]
    idx_v, rows_v, sem = refs[24], refs[25], refs[26]

    wid = lax.axis_index("s") * NC + lax.axis_index("c")
    base = wid * BPW

    for f in range(8):
        pltpu.sync_copy(idxs[f].at[pl.ds(wid * NCHUNK, NCHUNK)], idx_v)
        cps = []
        for j in range(NCHUNK):
            cps.append(
                pltpu.async_copy(
                    tabs[f].at[idx_v.at[j]],
                    rows_v.at[pl.ds(j * CHUNK, CHUNK)],
                    sem,
                )
            )
        for cp in cps:
            cp.wait()
        pltpu.sync_copy(rows_v, outs[f].at[pl.ds(base, BPW)])


def _make_sc_gather():
    mesh = plsc.VectorSubcoreMesh(core_axis_name="c", subcore_axis_name="s")
    out_type = tuple(
        jax.ShapeDtypeStruct((B, EMB), jnp.float32) for _ in range(8)
    )
    scratch = [
        pltpu.VMEM((NCHUNK, CHUNK), jnp.int32),
        pltpu.VMEM((BPW, EMB), jnp.float32),
        pltpu.SemaphoreType.DMA,
    ]
    return pl.kernel(
        _sc_gather_body,
        mesh=mesh,
        out_type=out_type,
        scratch_types=scratch,
        compiler_params=pltpu.CompilerParams(use_tc_tiling_on_sc=False),
    )


BLK = 2048


def _tc_dense_body(ue0, ue1, ue2, ue3, ie0, ie1, ie2, ie3,
                   uW1, ub1, uW2, ub2, iW1, ib1, iW2, ib2, oW, ob, out):
    uvec = jnp.concatenate([ue0[...], ue1[...], ue2[...], ue3[...]], axis=1)
    hu = jnp.maximum(
        jnp.dot(uvec, uW1[...], preferred_element_type=jnp.float32) + ub1[...], 0.0)
    u = jnp.maximum(
        jnp.dot(hu, uW2[...], preferred_element_type=jnp.float32) + ub2[...], 0.0)

    ivec = jnp.concatenate([ie0[...], ie1[...], ie2[...], ie3[...]], axis=1)
    hi = jnp.maximum(
        jnp.dot(ivec, iW1[...], preferred_element_type=jnp.float32) + ib1[...], 0.0)
    it = jnp.maximum(
        jnp.dot(hi, iW2[...], preferred_element_type=jnp.float32) + ib2[...], 0.0)

    dot = jnp.sum(u * it, axis=1, keepdims=True)
    nu = jnp.sqrt(jnp.sum(u * u, axis=1, keepdims=True))
    ni = jnp.sqrt(jnp.sum(it * it, axis=1, keepdims=True))
    cos = dot / (nu * ni + 1e-7)
    z = cos * oW[0, 0] + ob[0, 0]
    out[...] = 1.0 / (1.0 + jnp.exp(-z))


def _tc_dense(ue, ie, uW1, ub1, uW2, ub2, iW1, ib1, iW2, ib2, oW, ob):
    grid = (B // BLK,)
    emb_spec = pl.BlockSpec((BLK, EMB), lambda i: (i, 0))
    full = lambda shape: pl.BlockSpec(shape, lambda i: (0, 0))
    return pl.pallas_call(
        _tc_dense_body,
        grid=grid,
        in_specs=[emb_spec] * 8 + [
            full((128, 128)), full((1, 128)),
            full((128, 64)), full((1, 64)),
            full((128, 128)), full((1, 128)),
            full((128, 64)), full((1, 64)),
            full((1, 1)), full((1, 1)),
        ],
        out_specs=pl.BlockSpec((BLK, 1), lambda i: (i, 0)),
        out_shape=jax.ShapeDtypeStruct((B, 1), jnp.float32),
    )(*ue, *ie, uW1, ub1, uW2, ub2, iW1, ib1, iW2, ib2, oW, ob)


def kernel(user_id_idx, user_age_idx, user_gender_idx, user_city_idx,
           item_id_idx, item_cate_idx, item_brand_idx, item_price_idx,
           user_emb_user_id, user_emb_user_age, user_emb_user_gender, user_emb_user_city,
           item_emb_item_id, item_emb_item_cate, item_emb_item_brand, item_emb_item_price,
           uW1, ub1, uW2, ub2, iW1, ib1, iW2, ib2, oW, ob):
    tabs = (user_emb_user_id, user_emb_user_age, user_emb_user_gender, user_emb_user_city,
            item_emb_item_id, item_emb_item_cate, item_emb_item_brand, item_emb_item_price)
    idxs = tuple(
        a.astype(jnp.int32).reshape(IDX_ROWS, CHUNK)
        for a in (user_id_idx, user_age_idx, user_gender_idx, user_city_idx,
                  item_id_idx, item_cate_idx, item_brand_idx, item_price_idx)
    )
    gathered = _make_sc_gather()(*tabs, *idxs)
    return _tc_dense(
        gathered[:4], gathered[4:],
        uW1, ub1.reshape(1, 128), uW2, ub2.reshape(1, 64),
        iW1, ib1.reshape(1, 128), iW2, ib2.reshape(1, 64),
        oW, ob.reshape(1, 1),
    )
